# Initial kernel scaffold; baseline (speedup 1.0000x reference)
import jax
import jax.numpy as jnp
from jax.experimental import pallas as pl


def kernel(positions, hidden_states, Wq, Wk, Wv, Wo, q_norm_scale, k_norm_scale, input_ln_scale, post_ln_scale, Wg, W_gate, W_up, W_down):
    raise NotImplementedError("write your pallas kernel here")


# speedup vs baseline: 1.1331x; 1.1331x over previous
import jax
import jax.numpy as jnp
from jax.experimental import pallas as pl
from jax.experimental.pallas import tpu as pltpu

T = 2048
D = 1024
NH = 16
NKV = 4
HD = 128
E = 8
TOPK = 2
DFF = 768
EPS = 1e-6
THETA = 1000000.0
HALF = HD // 2


def _rms_norm(x, scale):
    var = jnp.mean(jnp.square(x), axis=-1, keepdims=True)
    return x * jax.lax.rsqrt(var + EPS) * scale


def _apply_rope(x, positions):
    inv_freq = 1.0 / (THETA ** (jnp.arange(0, HALF, dtype=jnp.float32) / HALF))
    freqs = positions.astype(jnp.float32)[:, None] * inv_freq[None, :]
    cos = jnp.cos(freqs)[:, None, :]
    sin = jnp.sin(freqs)[:, None, :]
    x1 = x[..., :HALF]
    x2 = x[..., HALF:]
    return jnp.concatenate([x1 * cos - x2 * sin, x2 * cos + x1 * sin], axis=-1)


BTM = 512


def _moe_body(h2_ref, comb_ref, wg_ref, wu_ref, wd_ref, out_ref):
    e = pl.program_id(1)

    @pl.when(e == 0)
    def _():
        out_ref[...] = jnp.zeros_like(out_ref)

    h2 = h2_ref[...]
    col = jax.lax.broadcasted_iota(jnp.int32, (BTM, 128), 1)
    c = jnp.sum(jnp.where(col == e, comb_ref[...], 0.0), axis=1, keepdims=True)
    g = jnp.dot(h2, wg_ref[0], preferred_element_type=jnp.float32)
    u = jnp.dot(h2, wu_ref[0], preferred_element_type=jnp.float32)
    a = (g / (1.0 + jnp.exp(-g))) * u
    y = jnp.dot(a, wd_ref[0], preferred_element_type=jnp.float32)
    out_ref[...] += c * y


def _moe(h2, comb, W_gate, W_up, W_down):
    grid = (T // BTM, E)
    return pl.pallas_call(
        _moe_body,
        grid=grid,
        in_specs=[
            pl.BlockSpec((BTM, D), lambda t, e: (t, 0)),
            pl.BlockSpec((BTM, 128), lambda t, e: (t, 0)),
            pl.BlockSpec((1, D, DFF), lambda t, e: (e, 0, 0)),
            pl.BlockSpec((1, D, DFF), lambda t, e: (e, 0, 0)),
            pl.BlockSpec((1, DFF, D), lambda t, e: (e, 0, 0)),
        ],
        out_specs=pl.BlockSpec((BTM, D), lambda t, e: (t, 0)),
        out_shape=jax.ShapeDtypeStruct((T, D), jnp.float32),
        compiler_params=pltpu.CompilerParams(
            dimension_semantics=("parallel", "arbitrary")),
    )(h2, comb, W_gate, W_up, W_down)


def kernel(positions, hidden_states, Wq, Wk, Wv, Wo, q_norm_scale,
           k_norm_scale, input_ln_scale, post_ln_scale, Wg, W_gate, W_up,
           W_down):
    residual = hidden_states
    h = _rms_norm(hidden_states, input_ln_scale)
    q = (h @ Wq).reshape(T, NH, HD)
    k = (h @ Wk).reshape(T, NKV, HD)
    v = (h @ Wv).reshape(T, NKV, HD)
    q = _rms_norm(q, q_norm_scale)
    k = _rms_norm(k, k_norm_scale)
    q = _apply_rope(q, positions)
    k = _apply_rope(k, positions)
    rep = NH // NKV
    k = jnp.repeat(k, rep, axis=1)
    v = jnp.repeat(v, rep, axis=1)
    qh = q.transpose(1, 0, 2)
    kh = k.transpose(1, 0, 2)
    vh = v.transpose(1, 0, 2)
    scores = jnp.einsum('htd,hsd->hts', qh, kh) * (HD ** -0.5)
    causal = jnp.tril(jnp.ones((T, T), dtype=bool))
    scores = jnp.where(causal[None, :, :], scores, jnp.float32(-1e30))
    probs = jax.nn.softmax(scores, axis=-1)
    attn = jnp.einsum('hts,hsd->htd', probs, vh)
    attn = attn.transpose(1, 0, 2).reshape(T, NH * HD)
    attn_out = attn @ Wo
    h2r = attn_out + residual
    residual2 = h2r
    h2 = _rms_norm(h2r, post_ln_scale)
    router_logits = h2 @ Wg
    router_probs = jax.nn.softmax(router_logits.astype(jnp.float32), axis=-1)
    topk_w, topk_idx = jax.lax.top_k(router_probs, TOPK)
    topk_w = topk_w / jnp.sum(topk_w, axis=-1, keepdims=True)
    combine = jnp.zeros((T, E), dtype=jnp.float32).at[
        jnp.arange(T)[:, None], topk_idx].add(topk_w)
    comb_pad = jnp.pad(combine, ((0, 0), (0, 128 - E)))

    out = _moe(h2, comb_pad, W_gate, W_up, W_down)
    return (out, residual2)


# speedup vs baseline: 1.2683x; 1.1194x over previous
import functools

import jax
import jax.numpy as jnp
from jax import lax
from jax.experimental import pallas as pl
from jax.experimental.pallas import tpu as pltpu
from jax.experimental.pallas import tpu_sc as plsc

T = 2048
D = 1024
NH = 16
NKV = 4
HD = 128
E = 8
TOPK = 2
DFF = 768
EPS = 1e-6
THETA = 1000000.0
HALF = HD // 2

BLK = 256
NBMAX = 22
RMAX = NBMAX * BLK
NW = 32
CHUNK = T // NW


def _rms_norm(x, scale):
    var = jnp.mean(jnp.square(x), axis=-1, keepdims=True)
    return x * jax.lax.rsqrt(var + EPS) * scale


def _apply_rope(x, positions):
    inv_freq = 1.0 / (THETA ** (jnp.arange(0, HALF, dtype=jnp.float32) / HALF))
    freqs = positions.astype(jnp.float32)[:, None] * inv_freq[None, :]
    cos = jnp.cos(freqs)[:, None, :]
    sin = jnp.sin(freqs)[:, None, :]
    x1 = x[..., :HALF]
    x2 = x[..., HALF:]
    return jnp.concatenate([x1 * cos - x2 * sin, x2 * cos + x1 * sin], axis=-1)


def _route_body(a_ref, pos_ref, be_ref):
    cols = jax.lax.broadcasted_iota(jnp.int32, (BLK, 128), 1)
    tri = (jax.lax.broadcasted_iota(jnp.int32, (BLK, BLK), 0)
           > jax.lax.broadcasted_iota(jnp.int32, (BLK, BLK), 1)
           ).astype(jnp.float32)
    nblk = (2 * T) // BLK
    counts = jnp.zeros((1, 128), jnp.float32)
    for b in range(nblk):
        o = (a_ref[b * BLK:(b + 1) * BLK, :] == cols).astype(jnp.float32)
        counts = counts + jnp.sum(o, axis=0, keepdims=True)
    padded = jnp.floor((counts + (BLK - 1)) * (1.0 / BLK)) * BLK
    etri = (jax.lax.broadcasted_iota(jnp.int32, (128, 128), 0)
            < jax.lax.broadcasted_iota(jnp.int32, (128, 128), 1)
            ).astype(jnp.float32)
    off = jnp.dot(padded, etri, preferred_element_type=jnp.float32)
    bstart = (jax.lax.broadcasted_iota(jnp.int32, (128, 128), 0)
              ).astype(jnp.float32) * BLK
    offb = jnp.broadcast_to(off, (128, 128))
    endb = offb + jnp.broadcast_to(padded, (128, 128))
    ind = ((bstart >= offb) & (bstart < endb)).astype(jnp.float32)
    evals = jnp.broadcast_to(
        jax.lax.broadcasted_iota(jnp.int32, (1, 128), 1).astype(jnp.float32),
        (128, 128))
    be = jnp.sum(ind * evals, axis=1, keepdims=True)
    valid = jnp.sum(ind, axis=1, keepdims=True)
    be_ref[...] = jnp.where(valid > 0.0, be, -1.0).astype(jnp.int32)
    carry = jnp.zeros((1, 128), jnp.float32)
    for b in range(nblk):
        o = (a_ref[b * BLK:(b + 1) * BLK, :] == cols).astype(jnp.float32)
        scum = jnp.dot(tri, o, preferred_element_type=jnp.float32)
        base = jnp.broadcast_to(carry + off, (BLK, 128))
        posb = jnp.sum(o * (base + scum), axis=1, keepdims=True)
        pos_ref[b * BLK:(b + 1) * BLK, :] = posb.astype(jnp.int32)
        carry = carry + jnp.sum(o, axis=0, keepdims=True)


def _route(a):
    return pl.pallas_call(
        _route_body,
        in_specs=[pl.BlockSpec((2 * T, 1), lambda: (0, 0))],
        out_specs=[
            pl.BlockSpec((2 * T, 1), lambda: (0, 0)),
            pl.BlockSpec((128, 1), lambda: (0, 0)),
        ],
        out_shape=[
            jax.ShapeDtypeStruct((2 * T, 1), jnp.int32),
            jax.ShapeDtypeStruct((128, 1), jnp.int32),
        ],
    )(a)


@functools.partial(
    pl.kernel,
    mesh=plsc.VectorSubcoreMesh(core_axis_name="c", subcore_axis_name="s"),
    out_type=jax.ShapeDtypeStruct((RMAX, D), jnp.float32),
    scratch_types=[
        pltpu.VMEM((CHUNK, D), jnp.float32),
        pltpu.VMEM((2, CHUNK), jnp.int32),
        pltpu.SemaphoreType.DMA,
    ],
)
def _sc_dispatch(h2_hbm, pos3_hbm, x_hbm, rows_v, idx_v, sem):
    wid = lax.axis_index("s") * 2 + lax.axis_index("c")
    pltpu.sync_copy(pos3_hbm.at[wid], idx_v)
    pltpu.sync_copy(h2_hbm.at[pl.ds(wid * CHUNK, CHUNK)], rows_v)
    pltpu.async_copy(rows_v, x_hbm.at[idx_v.at[0]], sem).wait()
    pltpu.async_copy(rows_v, x_hbm.at[idx_v.at[1]], sem).wait()


def _ffn_body(be_sref, x_ref, wg_ref, wu_ref, wd_ref, y_ref):
    b = pl.program_id(0)

    @pl.when(be_sref[b] >= 0)
    def _():
        x = x_ref[...]
        g = jnp.dot(x, wg_ref[0], preferred_element_type=jnp.float32)
        u = jnp.dot(x, wu_ref[0], preferred_element_type=jnp.float32)
        a = (g / (1.0 + jnp.exp(-g))) * u
        y_ref[...] = jnp.dot(a, wd_ref[0], preferred_element_type=jnp.float32)

    @pl.when(be_sref[b] < 0)
    def _():
        y_ref[...] = jnp.zeros_like(y_ref)


def _ffn(be, x, W_gate, W_up, W_down):
    grid_spec = pltpu.PrefetchScalarGridSpec(
        num_scalar_prefetch=1,
        grid=(NBMAX,),
        in_specs=[
            pl.BlockSpec((BLK, D), lambda b, be: (b, 0)),
            pl.BlockSpec((1, D, DFF), lambda b, be: (jnp.maximum(be[b], 0), 0, 0)),
            pl.BlockSpec((1, D, DFF), lambda b, be: (jnp.maximum(be[b], 0), 0, 0)),
            pl.BlockSpec((1, DFF, D), lambda b, be: (jnp.maximum(be[b], 0), 0, 0)),
        ],
        out_specs=pl.BlockSpec((BLK, D), lambda b, be: (b, 0)),
    )
    return pl.pallas_call(
        _ffn_body,
        grid_spec=grid_spec,
        out_shape=jax.ShapeDtypeStruct((RMAX, D), jnp.float32),
        compiler_params=pltpu.CompilerParams(
            dimension_semantics=("arbitrary",)),
    )(be, x, W_gate, W_up, W_down)


@functools.partial(
    pl.kernel,
    mesh=plsc.VectorSubcoreMesh(core_axis_name="c", subcore_axis_name="s"),
    out_type=[
        jax.ShapeDtypeStruct((T, D), jnp.float32),
        jax.ShapeDtypeStruct((T, D), jnp.float32),
    ],
    scratch_types=[
        pltpu.VMEM((CHUNK, D), jnp.float32),
        pltpu.VMEM((2, CHUNK), jnp.int32),
        pltpu.SemaphoreType.DMA,
    ],
)
def _sc_combine(y_hbm, pos3_hbm, y0_hbm, y1_hbm, rows_v, idx_v, sem):
    wid = lax.axis_index("s") * 2 + lax.axis_index("c")
    pltpu.sync_copy(pos3_hbm.at[wid], idx_v)
    pltpu.async_copy(y_hbm.at[idx_v.at[0]], rows_v, sem).wait()
    pltpu.sync_copy(rows_v, y0_hbm.at[pl.ds(wid * CHUNK, CHUNK)])
    pltpu.async_copy(y_hbm.at[idx_v.at[1]], rows_v, sem).wait()
    pltpu.sync_copy(rows_v, y1_hbm.at[pl.ds(wid * CHUNK, CHUNK)])


def _mix_body(y0_ref, y1_ref, w0_ref, w1_ref, out_ref):
    out_ref[...] = w0_ref[...] * y0_ref[...] + w1_ref[...] * y1_ref[...]


def _mix(y0, y1, w0, w1):
    BTX = 512
    return pl.pallas_call(
        _mix_body,
        grid=(T // BTX,),
        in_specs=[
            pl.BlockSpec((BTX, D), lambda i: (i, 0)),
            pl.BlockSpec((BTX, D), lambda i: (i, 0)),
            pl.BlockSpec((BTX, 1), lambda i: (i, 0)),
            pl.BlockSpec((BTX, 1), lambda i: (i, 0)),
        ],
        out_specs=pl.BlockSpec((BTX, D), lambda i: (i, 0)),
        out_shape=jax.ShapeDtypeStruct((T, D), jnp.float32),
        compiler_params=pltpu.CompilerParams(
            dimension_semantics=("parallel",)),
    )(y0, y1, w0, w1)


def kernel(positions, hidden_states, Wq, Wk, Wv, Wo, q_norm_scale,
           k_norm_scale, input_ln_scale, post_ln_scale, Wg, W_gate, W_up,
           W_down):
    residual = hidden_states
    h = _rms_norm(hidden_states, input_ln_scale)
    q = (h @ Wq).reshape(T, NH, HD)
    k = (h @ Wk).reshape(T, NKV, HD)
    v = (h @ Wv).reshape(T, NKV, HD)
    q = _rms_norm(q, q_norm_scale)
    k = _rms_norm(k, k_norm_scale)
    q = _apply_rope(q, positions)
    k = _apply_rope(k, positions)
    rep = NH // NKV
    k = jnp.repeat(k, rep, axis=1)
    v = jnp.repeat(v, rep, axis=1)
    qh = q.transpose(1, 0, 2)
    kh = k.transpose(1, 0, 2)
    vh = v.transpose(1, 0, 2)
    scores = jnp.einsum('htd,hsd->hts', qh, kh) * (HD ** -0.5)
    causal = jnp.tril(jnp.ones((T, T), dtype=bool))
    scores = jnp.where(causal[None, :, :], scores, jnp.float32(-1e30))
    probs = jax.nn.softmax(scores, axis=-1)
    attn = jnp.einsum('hts,hsd->htd', probs, vh)
    attn = attn.transpose(1, 0, 2).reshape(T, NH * HD)
    attn_out = attn @ Wo
    h2r = attn_out + residual
    residual2 = h2r
    h2 = _rms_norm(h2r, post_ln_scale)
    router_logits = h2 @ Wg
    router_probs = jax.nn.softmax(router_logits.astype(jnp.float32), axis=-1)
    topk_w, topk_idx = jax.lax.top_k(router_probs, TOPK)
    topk_w = topk_w / jnp.sum(topk_w, axis=-1, keepdims=True)

    a = topk_idx.transpose(1, 0).reshape(2 * T, 1).astype(jnp.int32)
    pos, be = _route(a)
    pos3 = pos.reshape(2, NW, CHUNK).transpose(1, 0, 2)
    x_sorted = _sc_dispatch(h2, pos3)
    y = _ffn(be.reshape(128)[:NBMAX], x_sorted, W_gate, W_up, W_down)
    y0, y1 = _sc_combine(y, pos3)
    out = _mix(y0, y1, topk_w[:, 0:1], topk_w[:, 1:2])
    return (out, residual2)


# speedup vs baseline: 1.2690x; 1.0005x over previous
import functools

import jax
import jax.numpy as jnp
from jax import lax
from jax.experimental import pallas as pl
from jax.experimental.pallas import tpu as pltpu
from jax.experimental.pallas import tpu_sc as plsc

T = 2048
D = 1024
NH = 16
NKV = 4
HD = 128
E = 8
TOPK = 2
DFF = 768
EPS = 1e-6
THETA = 1000000.0
HALF = HD // 2

BLK = 256
NBMAX = 22
RMAX = NBMAX * BLK
NW = 32
CHUNK = T // NW


def _rms_norm(x, scale):
    var = jnp.mean(jnp.square(x), axis=-1, keepdims=True)
    return x * jax.lax.rsqrt(var + EPS) * scale


def _apply_rope(x, positions):
    inv_freq = 1.0 / (THETA ** (jnp.arange(0, HALF, dtype=jnp.float32) / HALF))
    freqs = positions.astype(jnp.float32)[:, None] * inv_freq[None, :]
    cos = jnp.cos(freqs)[:, None, :]
    sin = jnp.sin(freqs)[:, None, :]
    x1 = x[..., :HALF]
    x2 = x[..., HALF:]
    return jnp.concatenate([x1 * cos - x2 * sin, x2 * cos + x1 * sin], axis=-1)


def _route_body(a_ref, pos_ref, be_ref):
    cols = jax.lax.broadcasted_iota(jnp.int32, (BLK, 128), 1)
    tri = (jax.lax.broadcasted_iota(jnp.int32, (BLK, BLK), 0)
           > jax.lax.broadcasted_iota(jnp.int32, (BLK, BLK), 1)
           ).astype(jnp.float32)
    nblk = (2 * T) // BLK
    counts = jnp.zeros((1, 128), jnp.float32)
    for b in range(nblk):
        o = (a_ref[b * BLK:(b + 1) * BLK, :] == cols).astype(jnp.float32)
        counts = counts + jnp.sum(o, axis=0, keepdims=True)
    padded = jnp.floor((counts + (BLK - 1)) * (1.0 / BLK)) * BLK
    etri = (jax.lax.broadcasted_iota(jnp.int32, (128, 128), 0)
            < jax.lax.broadcasted_iota(jnp.int32, (128, 128), 1)
            ).astype(jnp.float32)
    off = jnp.dot(padded, etri, preferred_element_type=jnp.float32)
    bstart = (jax.lax.broadcasted_iota(jnp.int32, (128, 128), 0)
              ).astype(jnp.float32) * BLK
    offb = jnp.broadcast_to(off, (128, 128))
    endb = offb + jnp.broadcast_to(padded, (128, 128))
    ind = ((bstart >= offb) & (bstart < endb)).astype(jnp.float32)
    evals = jnp.broadcast_to(
        jax.lax.broadcasted_iota(jnp.int32, (1, 128), 1).astype(jnp.float32),
        (128, 128))
    be = jnp.sum(ind * evals, axis=1, keepdims=True)
    valid = jnp.sum(ind, axis=1, keepdims=True)
    be_ref[...] = jnp.where(valid > 0.0, be, -1.0).astype(jnp.int32)
    carry = jnp.zeros((1, 128), jnp.float32)
    for b in range(nblk):
        o = (a_ref[b * BLK:(b + 1) * BLK, :] == cols).astype(jnp.float32)
        scum = jnp.dot(tri, o, preferred_element_type=jnp.float32)
        base = jnp.broadcast_to(carry + off, (BLK, 128))
        posb = jnp.sum(o * (base + scum), axis=1, keepdims=True)
        pos_ref[b * BLK:(b + 1) * BLK, :] = posb.astype(jnp.int32)
        carry = carry + jnp.sum(o, axis=0, keepdims=True)


def _route(a):
    return pl.pallas_call(
        _route_body,
        in_specs=[pl.BlockSpec((2 * T, 1), lambda: (0, 0))],
        out_specs=[
            pl.BlockSpec((2 * T, 1), lambda: (0, 0)),
            pl.BlockSpec((128, 1), lambda: (0, 0)),
        ],
        out_shape=[
            jax.ShapeDtypeStruct((2 * T, 1), jnp.int32),
            jax.ShapeDtypeStruct((128, 1), jnp.int32),
        ],
    )(a)


@functools.partial(
    pl.kernel,
    mesh=plsc.VectorSubcoreMesh(core_axis_name="c", subcore_axis_name="s"),
    out_type=jax.ShapeDtypeStruct((RMAX, D), jnp.float32),
    scratch_types=[
        pltpu.VMEM((CHUNK, D), jnp.float32),
        pltpu.VMEM((2, CHUNK), jnp.int32),
        pltpu.SemaphoreType.DMA,
    ],
)
def _sc_dispatch(h2_hbm, pos2_hbm, x_hbm, rows_v, idx_v, sem):
    wid = lax.axis_index("s") * 2 + lax.axis_index("c")
    pltpu.sync_copy(pos2_hbm.at[0].at[pl.ds(wid * CHUNK, CHUNK)], idx_v.at[0])
    pltpu.sync_copy(pos2_hbm.at[1].at[pl.ds(wid * CHUNK, CHUNK)], idx_v.at[1])
    pltpu.sync_copy(h2_hbm.at[pl.ds(wid * CHUNK, CHUNK)], rows_v)
    pltpu.async_copy(rows_v, x_hbm.at[idx_v.at[0]], sem).wait()
    pltpu.async_copy(rows_v, x_hbm.at[idx_v.at[1]], sem).wait()


def _ffn_body(be_sref, x_ref, wg_ref, wu_ref, wd_ref, y_ref):
    b = pl.program_id(0)

    @pl.when(be_sref[b] >= 0)
    def _():
        x = x_ref[...]
        g = jnp.dot(x, wg_ref[0], preferred_element_type=jnp.float32)
        u = jnp.dot(x, wu_ref[0], preferred_element_type=jnp.float32)
        a = (g / (1.0 + jnp.exp(-g))) * u
        y_ref[...] = jnp.dot(a, wd_ref[0], preferred_element_type=jnp.float32)

    @pl.when(be_sref[b] < 0)
    def _():
        y_ref[...] = jnp.zeros_like(y_ref)


def _ffn(be, x, W_gate, W_up, W_down):
    grid_spec = pltpu.PrefetchScalarGridSpec(
        num_scalar_prefetch=1,
        grid=(NBMAX,),
        in_specs=[
            pl.BlockSpec((BLK, D), lambda b, be: (b, 0)),
            pl.BlockSpec((1, D, DFF), lambda b, be: (jnp.maximum(be[b], 0), 0, 0)),
            pl.BlockSpec((1, D, DFF), lambda b, be: (jnp.maximum(be[b], 0), 0, 0)),
            pl.BlockSpec((1, DFF, D), lambda b, be: (jnp.maximum(be[b], 0), 0, 0)),
        ],
        out_specs=pl.BlockSpec((BLK, D), lambda b, be: (b, 0)),
    )
    return pl.pallas_call(
        _ffn_body,
        grid_spec=grid_spec,
        out_shape=jax.ShapeDtypeStruct((RMAX, D), jnp.float32),
        compiler_params=pltpu.CompilerParams(
            dimension_semantics=("arbitrary",)),
    )(be, x, W_gate, W_up, W_down)


@functools.partial(
    pl.kernel,
    mesh=plsc.VectorSubcoreMesh(core_axis_name="c", subcore_axis_name="s"),
    out_type=[
        jax.ShapeDtypeStruct((T, D), jnp.float32),
        jax.ShapeDtypeStruct((T, D), jnp.float32),
    ],
    scratch_types=[
        pltpu.VMEM((CHUNK, D), jnp.float32),
        pltpu.VMEM((2, CHUNK), jnp.int32),
        pltpu.SemaphoreType.DMA,
    ],
)
def _sc_combine(y_hbm, pos2_hbm, y0_hbm, y1_hbm, rows_v, idx_v, sem):
    wid = lax.axis_index("s") * 2 + lax.axis_index("c")
    pltpu.sync_copy(pos2_hbm.at[0].at[pl.ds(wid * CHUNK, CHUNK)], idx_v.at[0])
    pltpu.sync_copy(pos2_hbm.at[1].at[pl.ds(wid * CHUNK, CHUNK)], idx_v.at[1])
    pltpu.async_copy(y_hbm.at[idx_v.at[0]], rows_v, sem).wait()
    pltpu.sync_copy(rows_v, y0_hbm.at[pl.ds(wid * CHUNK, CHUNK)])
    pltpu.async_copy(y_hbm.at[idx_v.at[1]], rows_v, sem).wait()
    pltpu.sync_copy(rows_v, y1_hbm.at[pl.ds(wid * CHUNK, CHUNK)])


def _mix_body(y0_ref, y1_ref, w0_ref, w1_ref, out_ref):
    out_ref[...] = w0_ref[...] * y0_ref[...] + w1_ref[...] * y1_ref[...]


def _mix(y0, y1, w0, w1):
    BTX = 512
    return pl.pallas_call(
        _mix_body,
        grid=(T // BTX,),
        in_specs=[
            pl.BlockSpec((BTX, D), lambda i: (i, 0)),
            pl.BlockSpec((BTX, D), lambda i: (i, 0)),
            pl.BlockSpec((BTX, 1), lambda i: (i, 0)),
            pl.BlockSpec((BTX, 1), lambda i: (i, 0)),
        ],
        out_specs=pl.BlockSpec((BTX, D), lambda i: (i, 0)),
        out_shape=jax.ShapeDtypeStruct((T, D), jnp.float32),
        compiler_params=pltpu.CompilerParams(
            dimension_semantics=("parallel",)),
    )(y0, y1, w0, w1)


def kernel(positions, hidden_states, Wq, Wk, Wv, Wo, q_norm_scale,
           k_norm_scale, input_ln_scale, post_ln_scale, Wg, W_gate, W_up,
           W_down):
    residual = hidden_states
    h = _rms_norm(hidden_states, input_ln_scale)
    q = (h @ Wq).reshape(T, NH, HD)
    k = (h @ Wk).reshape(T, NKV, HD)
    v = (h @ Wv).reshape(T, NKV, HD)
    q = _rms_norm(q, q_norm_scale)
    k = _rms_norm(k, k_norm_scale)
    q = _apply_rope(q, positions)
    k = _apply_rope(k, positions)
    rep = NH // NKV
    k = jnp.repeat(k, rep, axis=1)
    v = jnp.repeat(v, rep, axis=1)
    qh = q.transpose(1, 0, 2)
    kh = k.transpose(1, 0, 2)
    vh = v.transpose(1, 0, 2)
    scores = jnp.einsum('htd,hsd->hts', qh, kh) * (HD ** -0.5)
    causal = jnp.tril(jnp.ones((T, T), dtype=bool))
    scores = jnp.where(causal[None, :, :], scores, jnp.float32(-1e30))
    probs = jax.nn.softmax(scores, axis=-1)
    attn = jnp.einsum('hts,hsd->htd', probs, vh)
    attn = attn.transpose(1, 0, 2).reshape(T, NH * HD)
    attn_out = attn @ Wo
    h2r = attn_out + residual
    residual2 = h2r
    h2 = _rms_norm(h2r, post_ln_scale)
    router_logits = h2 @ Wg
    router_probs = jax.nn.softmax(router_logits.astype(jnp.float32), axis=-1)
    topk_w, topk_idx = jax.lax.top_k(router_probs, TOPK)
    topk_w = topk_w / jnp.sum(topk_w, axis=-1, keepdims=True)

    a = topk_idx.transpose(1, 0).reshape(2 * T, 1).astype(jnp.int32)
    pos, be = _route(a)
    pos2 = pos.reshape(2, T)
    x_sorted = _sc_dispatch(h2, pos2)
    y = _ffn(be.reshape(128)[:NBMAX], x_sorted, W_gate, W_up, W_down)
    y0, y1 = _sc_combine(y, pos2)
    out = _mix(y0, y1, topk_w[:, 0:1], topk_w[:, 1:2])
    return (out, residual2)
